# Initial kernel scaffold; baseline (speedup 1.0000x reference)
import jax
import jax.numpy as jnp
from jax.experimental import pallas as pl


def kernel(x, sel_weights, head_weights, base_weight):
    raise NotImplementedError("write your pallas kernel here")


# speedup vs baseline: 3.8578x; 3.8578x over previous
import functools

import jax
import jax.numpy as jnp
from jax.experimental import pallas as pl
import jax.experimental.pallas.tpu as pltpu

LENGTH = 2048
CHANNEL = 1024
N_SEL = 8
NUM_HEADS = 4
NUM_FREQ = 16
BATCH = 4
CHUNK = CHANNEL // N_SEL

ROWS = 256


def _kernel(x_ref, selw_ref, hw_ref, base_ref, out_ref, w_scratch):
    b = pl.program_id(0)
    kblk = pl.program_id(1)

    @pl.when(jnp.logical_and(b == 0, kblk == 0))
    def _compute_w():
        selw = selw_ref[...]
        hw = hw_ref[...]
        hw_sm = jax.nn.softmax(hw[0], axis=0)
        coeff = jnp.zeros((N_SEL, NUM_FREQ), jnp.float32)
        krow = jax.lax.broadcasted_iota(jnp.int32, (N_SEL, NUM_FREQ), 0)
        fidx = jax.lax.broadcasted_iota(jnp.int32, (NUM_FREQ, NUM_FREQ), 0)
        gidx = jax.lax.broadcasted_iota(jnp.int32, (NUM_FREQ, NUM_FREQ), 1)
        for h in range(NUM_HEADS):
            logits = selw[h] * jnp.float32(h + 1)
            nw = jax.nn.softmax(logits, axis=0)
            ng = nw[:, None]
            nf = nw[None, :]
            beats = (ng > nf) | ((ng == nf) & (fidx < gidx))
            rank = jnp.sum(beats.astype(jnp.int32), axis=0)
            onehot = (krow == rank[None, :]).astype(jnp.float32)
            coeff = coeff + hw_sm[h] * onehot * nw[None, :]
        w_scratch[...] = jnp.dot(coeff, base_ref[...],
                                 preferred_element_type=jnp.float32)

    for j in range(ROWS // CHUNK):
        wrow = w_scratch[kblk * (ROWS // CHUNK) + j, :]
        xsub = x_ref[0, pl.ds(j * CHUNK, CHUNK), :]
        out_ref[0, 0, 0, pl.ds(j * CHUNK, CHUNK)] = jnp.sum(
            xsub * wrow[None, :], axis=1)


@jax.jit
def kernel(x, sel_weights, head_weights, base_weight):
    grid = (BATCH, CHANNEL // ROWS)
    return pl.pallas_call(
        _kernel,
        grid=grid,
        in_specs=[
            pl.BlockSpec((1, ROWS, LENGTH), lambda b, k: (b, k, 0)),
            pl.BlockSpec((NUM_HEADS, NUM_FREQ), lambda b, k: (0, 0)),
            pl.BlockSpec((1, NUM_HEADS), lambda b, k: (0, 0)),
            pl.BlockSpec((NUM_FREQ, LENGTH), lambda b, k: (0, 0)),
        ],
        out_specs=pl.BlockSpec((1, 1, 1, ROWS), lambda b, k: (b, k, 0, 0)),
        out_shape=jax.ShapeDtypeStruct((BATCH, CHANNEL // ROWS, 1, ROWS),
                                       jnp.float32),
        scratch_shapes=[pltpu.VMEM((N_SEL, LENGTH), jnp.float32)],
    )(x, sel_weights, head_weights.reshape(1, NUM_HEADS),
      base_weight).reshape(BATCH, CHANNEL)


# speedup vs baseline: 4.8533x; 1.2581x over previous
import functools

import jax
import jax.numpy as jnp
from jax.experimental import pallas as pl
import jax.experimental.pallas.tpu as pltpu

LENGTH = 2048
CHANNEL = 1024
N_SEL = 8
NUM_HEADS = 4
NUM_FREQ = 16
BATCH = 4
CHUNK = CHANNEL // N_SEL

ROWS = 512


def _kernel(x_ref, selw_ref, hw_ref, base_ref, out_ref, w_scratch):
    b = pl.program_id(0)
    kblk = pl.program_id(1)

    @pl.when(jnp.logical_and(b == 0, kblk == 0))
    def _compute_w():
        selw = selw_ref[...]
        hw = hw_ref[...]
        hw_sm = jax.nn.softmax(hw[0], axis=0)
        coeff = jnp.zeros((N_SEL, NUM_FREQ), jnp.float32)
        krow = jax.lax.broadcasted_iota(jnp.int32, (N_SEL, NUM_FREQ), 0)
        fidx = jax.lax.broadcasted_iota(jnp.int32, (NUM_FREQ, NUM_FREQ), 0)
        gidx = jax.lax.broadcasted_iota(jnp.int32, (NUM_FREQ, NUM_FREQ), 1)
        for h in range(NUM_HEADS):
            logits = selw[h] * jnp.float32(h + 1)
            nw = jax.nn.softmax(logits, axis=0)
            ng = nw[:, None]
            nf = nw[None, :]
            beats = (ng > nf) | ((ng == nf) & (fidx < gidx))
            rank = jnp.sum(beats.astype(jnp.int32), axis=0)
            onehot = (krow == rank[None, :]).astype(jnp.float32)
            coeff = coeff + hw_sm[h] * onehot * nw[None, :]
        w_scratch[...] = jnp.dot(coeff, base_ref[...],
                                 preferred_element_type=jnp.float32)

    for j in range(ROWS // CHUNK):
        wrow = w_scratch[kblk * (ROWS // CHUNK) + j, :]
        xsub = x_ref[0, pl.ds(j * CHUNK, CHUNK), :]
        out_ref[0, 0, 0, pl.ds(j * CHUNK, CHUNK)] = jnp.sum(
            xsub * wrow[None, :], axis=1)


@jax.jit
def kernel(x, sel_weights, head_weights, base_weight):
    grid = (BATCH, CHANNEL // ROWS)
    return pl.pallas_call(
        _kernel,
        grid=grid,
        in_specs=[
            pl.BlockSpec((1, ROWS, LENGTH), lambda b, k: (b, k, 0)),
            pl.BlockSpec((NUM_HEADS, NUM_FREQ), lambda b, k: (0, 0)),
            pl.BlockSpec((1, NUM_HEADS), lambda b, k: (0, 0)),
            pl.BlockSpec((NUM_FREQ, LENGTH), lambda b, k: (0, 0)),
        ],
        out_specs=pl.BlockSpec((1, 1, 1, ROWS), lambda b, k: (b, k, 0, 0)),
        out_shape=jax.ShapeDtypeStruct((BATCH, CHANNEL // ROWS, 1, ROWS),
                                       jnp.float32),
        scratch_shapes=[pltpu.VMEM((N_SEL, LENGTH), jnp.float32)],
    )(x, sel_weights, head_weights.reshape(1, NUM_HEADS),
      base_weight).reshape(BATCH, CHANNEL)


# speedup vs baseline: 5.2960x; 1.0912x over previous
import functools

import jax
import jax.numpy as jnp
from jax.experimental import pallas as pl
import jax.experimental.pallas.tpu as pltpu

LENGTH = 2048
CHANNEL = 1024
N_SEL = 8
NUM_HEADS = 4
NUM_FREQ = 16
BATCH = 4
CHUNK = CHANNEL // N_SEL

ROWS = 1024


def _kernel(x_ref, selw_ref, hw_ref, base_ref, out_ref, w_scratch):
    b = pl.program_id(0)
    kblk = pl.program_id(1)

    @pl.when(jnp.logical_and(b == 0, kblk == 0))
    def _compute_w():
        selw = selw_ref[...]
        hw = hw_ref[...]
        hw_sm = jax.nn.softmax(hw[0], axis=0)
        coeff = jnp.zeros((N_SEL, NUM_FREQ), jnp.float32)
        krow = jax.lax.broadcasted_iota(jnp.int32, (N_SEL, NUM_FREQ), 0)
        fidx = jax.lax.broadcasted_iota(jnp.int32, (NUM_FREQ, NUM_FREQ), 0)
        gidx = jax.lax.broadcasted_iota(jnp.int32, (NUM_FREQ, NUM_FREQ), 1)
        for h in range(NUM_HEADS):
            logits = selw[h] * jnp.float32(h + 1)
            nw = jax.nn.softmax(logits, axis=0)
            ng = nw[:, None]
            nf = nw[None, :]
            beats = (ng > nf) | ((ng == nf) & (fidx < gidx))
            rank = jnp.sum(beats.astype(jnp.int32), axis=0)
            onehot = (krow == rank[None, :]).astype(jnp.float32)
            coeff = coeff + hw_sm[h] * onehot * nw[None, :]
        w_scratch[...] = jnp.dot(coeff, base_ref[...],
                                 preferred_element_type=jnp.float32)

    for j in range(ROWS // CHUNK):
        wrow = w_scratch[kblk * (ROWS // CHUNK) + j, :]
        xsub = x_ref[0, pl.ds(j * CHUNK, CHUNK), :]
        out_ref[0, 0, 0, pl.ds(j * CHUNK, CHUNK)] = jnp.sum(
            xsub * wrow[None, :], axis=1)


@jax.jit
def kernel(x, sel_weights, head_weights, base_weight):
    grid = (BATCH, CHANNEL // ROWS)
    return pl.pallas_call(
        _kernel,
        grid=grid,
        in_specs=[
            pl.BlockSpec((1, ROWS, LENGTH), lambda b, k: (b, k, 0)),
            pl.BlockSpec((NUM_HEADS, NUM_FREQ), lambda b, k: (0, 0)),
            pl.BlockSpec((1, NUM_HEADS), lambda b, k: (0, 0)),
            pl.BlockSpec((NUM_FREQ, LENGTH), lambda b, k: (0, 0)),
        ],
        out_specs=pl.BlockSpec((1, 1, 1, ROWS), lambda b, k: (b, k, 0, 0)),
        out_shape=jax.ShapeDtypeStruct((BATCH, CHANNEL // ROWS, 1, ROWS),
                                       jnp.float32),
        scratch_shapes=[pltpu.VMEM((N_SEL, LENGTH), jnp.float32)],
    )(x, sel_weights, head_weights.reshape(1, NUM_HEADS),
      base_weight).reshape(BATCH, CHANNEL)
